# baseline (speedup 1.0000x reference)
import jax
import jax.numpy as jnp
from jax.experimental import pallas as pl
from jax.experimental.pallas import tpu as pltpu

B = 32
V = 1_000_000
BLK = 8192

_ROTS_EVEN = (13, 15, 26, 6)
_ROTS_ODD = (17, 29, 16, 24)
_K0 = 0
_K1 = 42
_K2 = _K0 ^ _K1 ^ 0x1BD11BDA
_KS = (_K0, _K1, _K2)
_TINY = float(jnp.finfo(jnp.float32).tiny)
_NEG_INF = float("-inf")


def _rotl(x, r):
    return (x << jnp.uint32(r)) | (x >> jnp.uint32(32 - r))


def _threefry_bits(j):
    x0 = jnp.zeros_like(j) + jnp.uint32(_K0)
    x1 = j + jnp.uint32(_K1)
    for g in range(5):
        rots = _ROTS_EVEN if g % 2 == 0 else _ROTS_ODD
        for r in rots:
            x0 = x0 + x1
            x1 = _rotl(x1, r)
            x1 = x1 ^ x0
        x0 = x0 + jnp.uint32(_KS[(g + 1) % 3])
        x1 = x1 + jnp.uint32(_KS[(g + 2) % 3] + (g + 1))
    return x0 ^ x1


def _sample_kernel(logits_ref, out_ref, rmax_ref, ridx_ref):
    i = pl.program_id(0)
    nsteps = pl.num_programs(0)

    neg_inf = jnp.float32(_NEG_INF)
    tiny = jnp.float32(_TINY)

    @pl.when(i == 0)
    def _init():
        rmax_ref[...] = jnp.full((B, 1), neg_inf, jnp.float32)
        ridx_ref[...] = jnp.zeros((B, 1), jnp.int32)

    c0 = i * BLK
    row = jax.lax.broadcasted_iota(jnp.int32, (B, BLK), 0)
    col_local = jax.lax.broadcasted_iota(jnp.int32, (B, BLK), 1)
    col = col_local + c0
    j = (row * V + col).astype(jnp.uint32)

    bits = _threefry_bits(j)
    fb = (bits >> jnp.uint32(9)) | jnp.uint32(0x3F800000)
    floats = jax.lax.bitcast_convert_type(fb, jnp.float32) - jnp.float32(1.0)
    u = jnp.maximum(tiny, floats + tiny)
    g = -jnp.log(-jnp.log(u))

    v = g + logits_ref[...]
    v = jnp.where(col < V, v, neg_inf)

    bmax = jnp.max(v, axis=1, keepdims=True)
    bidx = jnp.min(jnp.where(v == bmax, col, jnp.int32(2**31 - 1)),
                   axis=1, keepdims=True)

    better = bmax > rmax_ref[...]
    rmax_ref[...] = jnp.where(better, bmax, rmax_ref[...])
    ridx_ref[...] = jnp.where(better, bidx, ridx_ref[...])

    @pl.when(i == nsteps - 1)
    def _done():
        out_ref[...] = ridx_ref[...]


@jax.jit
def kernel(logits):
    nsteps = pl.cdiv(V, BLK)
    out = pl.pallas_call(
        _sample_kernel,
        grid=(nsteps,),
        in_specs=[pl.BlockSpec((B, BLK), lambda i: (0, i))],
        out_specs=pl.BlockSpec((B, 1), lambda i: (0, 0)),
        out_shape=jax.ShapeDtypeStruct((B, 1), jnp.int32),
        scratch_shapes=[
            pltpu.VMEM((B, 1), jnp.float32),
            pltpu.VMEM((B, 1), jnp.int32),
        ],
    )(logits)
    return out[:, 0].astype(jnp.int64)


# speedup vs baseline: 1.2221x; 1.2221x over previous
import jax
import jax.numpy as jnp
from jax.experimental import pallas as pl
from jax.experimental.pallas import tpu as pltpu

B = 32
V = 1_000_000
BLK = 2048

_ROTS_EVEN = (13, 15, 26, 6)
_ROTS_ODD = (17, 29, 16, 24)
_K0 = 0
_K1 = 42
_K2 = _K0 ^ _K1 ^ 0x1BD11BDA
_KS = (_K0, _K1, _K2)
_TINY = float(jnp.finfo(jnp.float32).tiny)
_NEG_INF = float("-inf")


def _rotl(x, r):
    return (x << jnp.uint32(r)) | (x >> jnp.uint32(32 - r))


def _threefry_bits(x1):
    x0 = x1
    x1 = _rotl(x1, _ROTS_EVEN[0]) ^ x0
    for r in _ROTS_EVEN[1:]:
        x0 = x0 + x1
        x1 = _rotl(x1, r) ^ x0
    x0 = x0 + jnp.uint32(_KS[1])
    x1 = x1 + jnp.uint32(_KS[2] + 1)
    for g in range(1, 5):
        rots = _ROTS_EVEN if g % 2 == 0 else _ROTS_ODD
        for r in rots:
            x0 = x0 + x1
            x1 = _rotl(x1, r) ^ x0
        x0 = x0 + jnp.uint32(_KS[(g + 1) % 3])
        x1 = x1 + jnp.uint32(_KS[(g + 2) % 3] + (g + 1))
    return x0 ^ x1


def _sample_kernel(logits_ref, out_ref, rmax_ref, ridx_ref, base_ref):
    i = pl.program_id(0)
    nsteps = pl.num_programs(0)

    neg_inf = jnp.float32(_NEG_INF)
    tiny = jnp.float32(_TINY)
    col_local = jax.lax.broadcasted_iota(jnp.int32, (B, BLK), 1)

    @pl.when(i == 0)
    def _init():
        rmax_ref[...] = jnp.full((B, 1), neg_inf, jnp.float32)
        ridx_ref[...] = jnp.zeros((B, 1), jnp.int32)
        row = jax.lax.broadcasted_iota(jnp.int32, (B, BLK), 0)
        base_ref[...] = (row * V + col_local + _K1).astype(jnp.uint32)

    c0 = i * BLK
    x1 = base_ref[...] + c0.astype(jnp.uint32)

    bits = _threefry_bits(x1)
    fb = (bits >> jnp.uint32(9)) | jnp.uint32(0x3F800000)
    floats = jax.lax.bitcast_convert_type(fb, jnp.float32) - jnp.float32(1.0)
    u = jnp.maximum(tiny, floats + tiny)
    g = -jnp.log(-jnp.log(u))

    v = g + logits_ref[...]
    v = jnp.where(col_local < V - c0, v, neg_inf)

    bmax = jnp.max(v, axis=1, keepdims=True)
    bidx = jnp.min(jnp.where(v == bmax, col_local, jnp.int32(2**31 - 1)),
                   axis=1, keepdims=True)

    better = bmax > rmax_ref[...]
    rmax_ref[...] = jnp.where(better, bmax, rmax_ref[...])
    ridx_ref[...] = jnp.where(better, bidx + c0, ridx_ref[...])

    @pl.when(i == nsteps - 1)
    def _done():
        out_ref[...] = ridx_ref[...]


@jax.jit
def kernel(logits):
    nsteps = pl.cdiv(V, BLK)
    out = pl.pallas_call(
        _sample_kernel,
        grid=(nsteps,),
        in_specs=[pl.BlockSpec((B, BLK), lambda i: (0, i))],
        out_specs=pl.BlockSpec((B, 1), lambda i: (0, 0)),
        out_shape=jax.ShapeDtypeStruct((B, 1), jnp.int32),
        scratch_shapes=[
            pltpu.VMEM((B, 1), jnp.float32),
            pltpu.VMEM((B, 1), jnp.int32),
            pltpu.VMEM((B, BLK), jnp.uint32),
        ],
    )(logits)
    return out[:, 0].astype(jnp.int64)


# speedup vs baseline: 1.5110x; 1.2364x over previous
import jax
import jax.numpy as jnp
from jax.experimental import pallas as pl
from jax.experimental.pallas import tpu as pltpu

B = 32
V = 1_000_000
BLK = 8192
CHUNK = 2048

_ROTS_EVEN = (13, 15, 26, 6)
_ROTS_ODD = (17, 29, 16, 24)
_K0 = 0
_K1 = 42
_K2 = _K0 ^ _K1 ^ 0x1BD11BDA
_KS = (_K0, _K1, _K2)
_TINY = float(jnp.finfo(jnp.float32).tiny)
_NEG_INF = float("-inf")


def _rotl(x, r):
    return (x << jnp.uint32(r)) | (x >> jnp.uint32(32 - r))


def _threefry_bits(x1):
    x0 = x1
    x1 = _rotl(x1, _ROTS_EVEN[0]) ^ x0
    for r in _ROTS_EVEN[1:]:
        x0 = x0 + x1
        x1 = _rotl(x1, r) ^ x0
    x0 = x0 + jnp.uint32(_KS[1])
    x1 = x1 + jnp.uint32(_KS[2] + 1)
    for g in range(1, 5):
        rots = _ROTS_EVEN if g % 2 == 0 else _ROTS_ODD
        for r in rots:
            x0 = x0 + x1
            x1 = _rotl(x1, r) ^ x0
        x0 = x0 + jnp.uint32(_KS[(g + 1) % 3])
        x1 = x1 + jnp.uint32(_KS[(g + 2) % 3] + (g + 1))
    return x0 ^ x1


def _gumbel_from_bits(bits):
    tiny = jnp.float32(_TINY)
    fb = (bits >> jnp.uint32(9)) | jnp.uint32(0x3F800000)
    floats = jax.lax.bitcast_convert_type(fb, jnp.float32) - jnp.float32(1.0)
    u = jnp.maximum(tiny, floats + tiny)
    return -jnp.log(-jnp.log(u))


def _sample_kernel(logits_ref, out_ref, accv_ref, accj_ref, base_ref):
    i = pl.program_id(0)
    nsteps = pl.num_programs(0)
    neg_inf = jnp.float32(_NEG_INF)

    @pl.when(i == 0)
    def _init():
        accv_ref[...] = jnp.full((B, CHUNK), neg_inf, jnp.float32)
        accj_ref[...] = jnp.zeros((B, CHUNK), jnp.uint32)
        row = jax.lax.broadcasted_iota(jnp.int32, (B, CHUNK), 0)
        chunk_col = jax.lax.broadcasted_iota(jnp.int32, (B, CHUNK), 1)
        base_ref[...] = (row * V + chunk_col + _K1).astype(jnp.uint32)

    def accumulate(masked):
        c0 = i * BLK
        base = base_ref[...]
        accv = accv_ref[...]
        accj = accj_ref[...]
        for k in range(BLK // CHUNK):
            off = c0 + k * CHUNK
            jp = base + off.astype(jnp.uint32)
            v = _gumbel_from_bits(_threefry_bits(jp)) + \
                logits_ref[:, k * CHUNK:(k + 1) * CHUNK]
            if masked:
                chunk_col = jax.lax.broadcasted_iota(jnp.int32, (B, CHUNK), 1)
                v = jnp.where(chunk_col < V - off, v, neg_inf)
            better = v > accv
            accv = jnp.maximum(accv, v)
            accj = jnp.where(better, jp, accj)
        accv_ref[...] = accv
        accj_ref[...] = accj

    @pl.when(i < nsteps - 1)
    def _body():
        accumulate(masked=False)

    @pl.when(i == nsteps - 1)
    def _last():
        accumulate(masked=True)
        accv = accv_ref[...]
        m = jnp.max(accv, axis=1, keepdims=True)
        accj = accj_ref[...].astype(jnp.int32)
        jbest = jnp.min(
            jnp.where(accv == m, accj, jnp.int32(2**31 - 1)),
            axis=1, keepdims=True)
        row = jax.lax.broadcasted_iota(jnp.int32, (B, 1), 0)
        out_ref[...] = jbest - _K1 - row * V


@jax.jit
def kernel(logits):
    nsteps = pl.cdiv(V, BLK)
    out = pl.pallas_call(
        _sample_kernel,
        grid=(nsteps,),
        in_specs=[pl.BlockSpec((B, BLK), lambda i: (0, i))],
        out_specs=pl.BlockSpec((B, 1), lambda i: (0, 0)),
        out_shape=jax.ShapeDtypeStruct((B, 1), jnp.int32),
        scratch_shapes=[
            pltpu.VMEM((B, CHUNK), jnp.float32),
            pltpu.VMEM((B, CHUNK), jnp.uint32),
            pltpu.VMEM((B, CHUNK), jnp.uint32),
        ],
    )(logits)
    return out[:, 0].astype(jnp.int64)
